# Initial kernel scaffold; baseline (speedup 1.0000x reference)
import jax
import jax.numpy as jnp
from jax.experimental import pallas as pl


def kernel(x, edge_index, batch, W1, b1, W2, b2, W3, b3):
    raise NotImplementedError("write your pallas kernel here")


# speedup vs baseline: 18.3143x; 18.3143x over previous
import functools

import jax
import jax.numpy as jnp
from jax import lax
from jax.experimental import pallas as pl
from jax.experimental.pallas import tpu as pltpu
from jax.experimental.pallas import tpu_sc as plsc

F32 = jnp.float32

_N = 10000
_E = 320000
_D = 128
_G = 128
_NC = 2
_NS = 16
_NW = _NC * _NS
_EW = _E // _NW
_K = 100
_CH = _EW // _K
_NP = 10240
_RPT = _NP // _NS
_R = 1000
_NB = _N // _R


def _deg_body(dst_hbm, e0_hbm, znd_hbm, deg_hbm,
              id0, id1, rows, acc_sh, semj0, semj1):
    c = lax.axis_index("c")
    s = lax.axis_index("s")
    w = c * _NS + s
    pltpu.sync_copy(znd_hbm.at[pl.ds(s * _RPT, _RPT)],
                    acc_sh.at[pl.ds(s * _RPT, _RPT)])
    pltpu.sync_copy(e0_hbm, rows)
    plsc.subcore_barrier()

    pltpu.async_copy(dst_hbm.at[w, 0], id0, semj0)
    pltpu.async_copy(dst_hbm.at[w, 1], id1, semj1)
    npair = _CH // 2

    def body(i, carry):
        k0 = 2 * i
        k1 = k0 + 1
        pltpu.make_async_copy(dst_hbm.at[w, k0], id0, semj0).wait()
        pltpu.sync_copy(rows, acc_sh.at[id0.at[0]], add=True)

        @pl.when(i + 1 < npair)
        def _():
            pltpu.async_copy(dst_hbm.at[w, k0 + 2], id0, semj0)

        pltpu.make_async_copy(dst_hbm.at[w, k1], id1, semj1).wait()
        pltpu.sync_copy(rows, acc_sh.at[id1.at[0]], add=True)

        @pl.when(i + 1 < npair)
        def _():
            pltpu.async_copy(dst_hbm.at[w, k1 + 2], id1, semj1)

        return carry

    lax.fori_loop(0, npair, body, 0)
    plsc.subcore_barrier()
    pltpu.sync_copy(acc_sh.at[pl.ds(s * _RPT, _RPT)],
                    deg_hbm.at[c, pl.ds(s * _RPT, _RPT)])


def _spmm_body(hp_hbm, src_hbm, dst_hbm, znd_hbm, agg_hbm,
                 is0, is1, id0, id1, rows0, rows1, acc_sh,
                 sem0, sem1, semi0, semi1, semj0, semj1):
    c = lax.axis_index("c")
    s = lax.axis_index("s")
    w = c * _NS + s
    pltpu.sync_copy(znd_hbm.at[pl.ds(s * _RPT, _RPT)],
                    acc_sh.at[pl.ds(s * _RPT, _RPT)])
    plsc.subcore_barrier()

    pltpu.async_copy(src_hbm.at[w, 0], is0, semi0)
    pltpu.async_copy(dst_hbm.at[w, 0], id0, semj0)
    pltpu.async_copy(src_hbm.at[w, 1], is1, semi1)
    pltpu.async_copy(dst_hbm.at[w, 1], id1, semj1)
    pltpu.make_async_copy(src_hbm.at[w, 0], is0, semi0).wait()
    pltpu.async_copy(hp_hbm.at[is0.at[0]], rows0, sem0)

    npair = _CH // 2

    def body(i, carry):
        k0 = 2 * i
        k1 = k0 + 1
        pltpu.make_async_copy(hp_hbm.at[is0.at[0]], rows0, sem0).wait()
        pltpu.make_async_copy(src_hbm.at[w, k1], is1, semi1).wait()
        pltpu.async_copy(hp_hbm.at[is1.at[0]], rows1, sem1)
        pltpu.make_async_copy(dst_hbm.at[w, k0], id0, semj0).wait()
        pltpu.sync_copy(rows0, acc_sh.at[id0.at[0]], add=True)

        @pl.when(i + 1 < npair)
        def _():
            pltpu.async_copy(src_hbm.at[w, k0 + 2], is0, semi0)
            pltpu.async_copy(dst_hbm.at[w, k0 + 2], id0, semj0)

        pltpu.make_async_copy(hp_hbm.at[is1.at[0]], rows1, sem1).wait()
        pltpu.make_async_copy(dst_hbm.at[w, k1], id1, semj1).wait()
        pltpu.sync_copy(rows1, acc_sh.at[id1.at[0]], add=True)

        @pl.when(i + 1 < npair)
        def _():
            pltpu.make_async_copy(src_hbm.at[w, k0 + 2], is0, semi0).wait()
            pltpu.async_copy(hp_hbm.at[is0.at[0]], rows0, sem0)
            pltpu.async_copy(src_hbm.at[w, k0 + 3], is1, semi1)
            pltpu.async_copy(dst_hbm.at[w, k0 + 3], id1, semj1)

        return carry

    lax.fori_loop(0, npair, body, 0)
    plsc.subcore_barrier()
    pltpu.sync_copy(acc_sh.at[pl.ds(s * _RPT, _RPT)],
                    agg_hbm.at[c, pl.ds(s * _RPT, _RPT)])


@functools.lru_cache(maxsize=None)
def _sc_kernels():
    mesh = plsc.VectorSubcoreMesh(
        core_axis_name="c", subcore_axis_name="s",
        num_cores=_NC, num_subcores=_NS)
    deg_kernel = pl.kernel(
        _deg_body,
        out_type=jax.ShapeDtypeStruct((_NC, _NP, _D), F32),
        mesh=mesh,
        scratch_types=[
            pltpu.VMEM((1, _K), jnp.int32),
            pltpu.VMEM((1, _K), jnp.int32),
            pltpu.VMEM((_K, _D), F32),
            pltpu.VMEM_SHARED((_NP, _D), F32),
            pltpu.SemaphoreType.DMA,
            pltpu.SemaphoreType.DMA,
        ],
    )
    spmm_kernel = pl.kernel(
        _spmm_body,
        out_type=jax.ShapeDtypeStruct((_NC, _NP, _D), F32),
        mesh=mesh,
        scratch_types=[
            pltpu.VMEM((1, _K), jnp.int32),
            pltpu.VMEM((1, _K), jnp.int32),
            pltpu.VMEM((1, _K), jnp.int32),
            pltpu.VMEM((1, _K), jnp.int32),
            pltpu.VMEM((_K, _D), F32),
            pltpu.VMEM((_K, _D), F32),
            pltpu.VMEM_SHARED((_NP, _D), F32),
            pltpu.SemaphoreType.DMA,
            pltpu.SemaphoreType.DMA,
            pltpu.SemaphoreType.DMA,
            pltpu.SemaphoreType.DMA,
            pltpu.SemaphoreType.DMA,
            pltpu.SemaphoreType.DMA,
        ],
    )
    return deg_kernel, spmm_kernel


def _dot(a, b):
    return jnp.dot(a, b, preferred_element_type=F32,
                   precision=lax.Precision.HIGHEST)


def _tc1_body(dga_ref, dgb_ref, x_ref, w_ref, hp_ref, dis_ref):
    deg = dga_ref[0, :, :1] + dgb_ref[0, :, :1] + 1.0
    dis = 1.0 / jnp.sqrt(deg)
    h = _dot(x_ref[...], w_ref[...])
    hp_ref[...] = h * dis
    dis_ref[...] = dis


def _tc1(deg2, x, w1):
    return pl.pallas_call(
        _tc1_body,
        grid=(_NB,),
        in_specs=[
            pl.BlockSpec((1, _R, _D), lambda i: (0, i, 0)),
            pl.BlockSpec((1, _R, _D), lambda i: (1, i, 0)),
            pl.BlockSpec((_R, _D), lambda i: (i, 0)),
            pl.BlockSpec((_D, _D), lambda i: (0, 0)),
        ],
        out_specs=[
            pl.BlockSpec((_R, _D), lambda i: (i, 0)),
            pl.BlockSpec((_R, 1), lambda i: (i, 0)),
        ],
        out_shape=[
            jax.ShapeDtypeStruct((_N, _D), F32),
            jax.ShapeDtypeStruct((_N, 1), F32),
        ],
    )(deg2, deg2, x, w1)


def _tcmid_body(aa_ref, ab_ref, hp_ref, dis_ref, b_ref, w_ref, out_ref):
    dis = dis_ref[...]
    xn = jnp.maximum(
        dis * (aa_ref[0] + ab_ref[0] + hp_ref[...]) + b_ref[...], 0.0)
    out_ref[...] = _dot(xn, w_ref[...]) * dis


def _tcmid(agg, hp, dis, b, w):
    return pl.pallas_call(
        _tcmid_body,
        grid=(_NB,),
        in_specs=[
            pl.BlockSpec((1, _R, _D), lambda i: (0, i, 0)),
            pl.BlockSpec((1, _R, _D), lambda i: (1, i, 0)),
            pl.BlockSpec((_R, _D), lambda i: (i, 0)),
            pl.BlockSpec((_R, 1), lambda i: (i, 0)),
            pl.BlockSpec((1, _D), lambda i: (0, 0)),
            pl.BlockSpec((_D, _D), lambda i: (0, 0)),
        ],
        out_specs=pl.BlockSpec((_R, _D), lambda i: (i, 0)),
        out_shape=jax.ShapeDtypeStruct((_N, _D), F32),
    )(agg, agg, hp, dis, b, w)


def _tcfin_body(aa_ref, ab_ref, hp_ref, dis_ref, b_ref, btt_ref, btc_ref,
                out_ref, sums, cnt):
    i = pl.program_id(0)

    @pl.when(i == 0)
    def _():
        sums[...] = jnp.zeros_like(sums)
        cnt[...] = jnp.zeros_like(cnt)
        out_ref[...] = jnp.full_like(out_ref, -jnp.inf)

    dis = dis_ref[...]
    h3 = jnp.maximum(
        dis * (aa_ref[0] + ab_ref[0] + hp_ref[...]) + b_ref[...], 0.0)

    btt0 = btt_ref[0]
    iot = lax.broadcasted_iota(jnp.int32, (_G, _R), 0)
    oht = (iot == btt0).astype(F32)
    sums[...] += _dot(oht, h3)
    cnt[...] += jnp.sum(oht, axis=1, keepdims=True)

    btc = btc_ref[...]
    hs = h3
    sh = 1
    while sh < _R:
        bshift = jnp.concatenate(
            [jnp.full((sh, 1), -1, jnp.int32), btc[:_R - sh]], axis=0)
        hshift = jnp.concatenate(
            [jnp.full((sh, _D), -jnp.inf, F32), hs[:_R - sh]], axis=0)
        hs = jnp.maximum(hs, jnp.where(btc == bshift, hshift, -jnp.inf))
        sh *= 2
    btnext = jnp.concatenate(
        [btt0[:, 1:], jnp.full((1, 1), -1, jnp.int32)], axis=1)
    ohlast = ((iot == btt0) & (btt0 != btnext)).astype(F32)
    mxb = _dot(ohlast, hs)
    present = jnp.sum(ohlast, axis=1, keepdims=True) > 0.0
    cur = out_ref[:, _D:]
    out_ref[:, _D:] = jnp.maximum(cur, jnp.where(present, mxb, -jnp.inf))

    @pl.when(i == _NB - 1)
    def _():
        out_ref[:, :_D] = sums[...] / jnp.maximum(cnt[...], 1.0)


def _tcfin(agg, hp, dis, b, btt, btc):
    return pl.pallas_call(
        _tcfin_body,
        grid=(_NB,),
        in_specs=[
            pl.BlockSpec((1, _R, _D), lambda i: (0, i, 0)),
            pl.BlockSpec((1, _R, _D), lambda i: (1, i, 0)),
            pl.BlockSpec((_R, _D), lambda i: (i, 0)),
            pl.BlockSpec((_R, 1), lambda i: (i, 0)),
            pl.BlockSpec((1, _D), lambda i: (0, 0)),
            pl.BlockSpec((1, 1, _R), lambda i: (i, 0, 0)),
            pl.BlockSpec((_R, 1), lambda i: (i, 0)),
        ],
        out_specs=pl.BlockSpec((_G, 2 * _D), lambda i: (0, 0)),
        out_shape=jax.ShapeDtypeStruct((_G, 2 * _D), F32),
        scratch_shapes=[
            pltpu.VMEM((_G, _D), F32),
            pltpu.VMEM((_G, 1), F32),
        ],
    )(agg, agg, hp, dis, b, btt, btc)


def kernel(x, edge_index, batch, W1, b1, W2, b2, W3, b3):
    src4 = edge_index[0].astype(jnp.int32).reshape(_NW, _CH, 1, _K)
    dst4 = edge_index[1].astype(jnp.int32).reshape(_NW, _CH, 1, _K)
    btt = batch.astype(jnp.int32).reshape(_NB, 1, _R)
    btc = batch.astype(jnp.int32).reshape(_N, 1)
    znd = jnp.zeros((_NP, _D), F32)
    e0 = jnp.zeros((_K, _D), F32).at[:, 0].set(1.0)

    deg_kernel, spmm_kernel = _sc_kernels()
    deg2 = deg_kernel(dst4, e0, znd)
    hp1, dis = _tc1(deg2, x, W1)
    agg1 = spmm_kernel(hp1, src4, dst4, znd)
    hp2 = _tcmid(agg1, hp1, dis, b1.reshape(1, _D), W2)
    agg2 = spmm_kernel(hp2, src4, dst4, znd)
    hp3 = _tcmid(agg2, hp2, dis, b2.reshape(1, _D), W3)
    agg3 = spmm_kernel(hp3, src4, dst4, znd)
    return _tcfin(agg3, hp3, dis, b3.reshape(1, _D), btt, btc)
